# baseline (speedup 1.0000x reference)
import functools
import math

import jax
import jax.numpy as jnp
from jax import lax
from jax.experimental import pallas as pl
from jax.experimental.pallas import tpu as pltpu
from jax.experimental.pallas import tpu_sc as plsc

_NC = 2
_NS = 16
_NW = _NC * _NS


def _router_body(n_tokens, n_experts, capacity, k_top,
                 x_ref, wr_ref, slots_ref, gates_ref, valid_ref, aux_ref):
    f32 = jnp.float32
    xb = x_ref[...]
    wr = wr_ref[...]
    logits = jnp.dot(xb, wr, preferred_element_type=f32)
    m = jnp.max(logits, axis=1, keepdims=True)
    ex = jnp.exp(logits - m)
    probs = ex / jnp.sum(ex, axis=1, keepdims=True)

    eidx = lax.broadcasted_iota(jnp.int32, (n_tokens, n_experts), 1)
    m1 = jnp.max(probs, axis=1, keepdims=True)
    i1 = jnp.min(jnp.where(probs == m1, eidx, n_experts), axis=1,
                 keepdims=True)
    probs_m = jnp.where(eidx == i1, -jnp.inf, probs)
    m2 = jnp.max(probs_m, axis=1, keepdims=True)
    i2 = jnp.min(jnp.where(probs_m == m2, eidx, n_experts), axis=1,
                 keepdims=True)
    ssum = m1 + m2
    g1 = m1 / ssum
    g2 = m2 / ssum

    oh = (eidx == i1).astype(f32) + (eidx == i2).astype(f32)

    me = jnp.sum(probs, axis=0, keepdims=True) / n_tokens
    ce = jnp.sum(oh, axis=0, keepdims=True) / n_tokens
    aux_ref[...] = ((n_experts / k_top) * jnp.sum(me * ce)).reshape(1, 1)

    blk = 256
    cnt_rows = []
    for b in range(n_tokens // blk):
        r = lax.broadcasted_iota(jnp.int32, (blk, n_tokens), 0) + (b * blk)
        c = lax.broadcasted_iota(jnp.int32, (blk, n_tokens), 1)
        trib = (c < r).astype(f32)
        cnt_rows.append(jnp.dot(trib, oh, preferred_element_type=f32))
    cnt = jnp.concatenate(cnt_rows, axis=0)

    p1 = jnp.sum(jnp.where(eidx == i1, cnt, 0.0), axis=1, keepdims=True)
    p2 = jnp.sum(jnp.where(eidx == i2, cnt, 0.0), axis=1, keepdims=True)
    p1i = p1.astype(jnp.int32)
    p2i = p2.astype(jnp.int32)
    v1 = p1i < capacity
    v2 = p2i < capacity
    slot1 = jnp.where(v1, i1 * capacity + p1i, 0)
    slot2 = jnp.where(v2, i2 * capacity + p2i, 0)
    slots_ref[...] = jnp.concatenate([slot1, slot2], axis=1)
    gates_ref[...] = jnp.concatenate(
        [jnp.where(v1, g1, 0.0), jnp.where(v2, g2, 0.0)], axis=1)
    valid_ref[...] = jnp.concatenate(
        [v1.astype(jnp.int32), v2.astype(jnp.int32)], axis=1)


def _router(x_flat, Wr, capacity, k_top):
    n_tokens, _ = x_flat.shape
    n_experts = Wr.shape[1]
    body = functools.partial(_router_body, n_tokens, n_experts, capacity,
                             k_top)
    return pl.pallas_call(
        body,
        out_shape=[
            jax.ShapeDtypeStruct((n_tokens, k_top), jnp.int32),
            jax.ShapeDtypeStruct((n_tokens, k_top), jnp.float32),
            jax.ShapeDtypeStruct((n_tokens, k_top), jnp.int32),
            jax.ShapeDtypeStruct((1, 1), jnp.float32),
        ],
    )(x_flat, Wr)


def _make_scatter_src(n_pairs, n_slots):
    mesh = plsc.VectorSubcoreMesh(core_axis_name="c", subcore_axis_name="s")

    @functools.partial(
        pl.kernel,
        mesh=mesh,
        out_type=jax.ShapeDtypeStruct((n_slots,), jnp.int32),
        scratch_types=[
            pltpu.VMEM((n_pairs,), jnp.int32),
            pltpu.VMEM((n_pairs,), jnp.int32),
            pltpu.VMEM((n_slots,), jnp.int32),
        ],
        compiler_params=pltpu.CompilerParams(needs_layout_passes=False),
    )
    def k(slots_hbm, valid_hbm, out_hbm, slots_v, valid_v, src_v):
        wid = lax.axis_index("c") * _NS + lax.axis_index("s")

        @pl.when(wid == 0)
        def _():
            pltpu.sync_copy(slots_hbm, slots_v)
            pltpu.sync_copy(valid_hbm, valid_v)

            def zbody(i, carry):
                sl_ids = i * 16 + lax.broadcasted_iota(jnp.int32, (16,), 0)
                src_v[pl.ds(i * 16, 16)] = lax.rem(sl_ids, n_pairs // 2)
                return carry

            lax.fori_loop(0, n_slots // 16, zbody, 0)

            def sbody(j, carry):
                sl = slots_v[pl.ds(j * 16, 16)]
                vm = valid_v[pl.ds(j * 16, 16)] > 0
                pair_ids = j * 16 + lax.broadcasted_iota(jnp.int32, (16,), 0)
                toks = lax.shift_right_logical(pair_ids, 1)
                plsc.store_scatter(src_v, [sl], toks, mask=vm)
                return carry

            lax.fori_loop(0, n_pairs // 16, sbody, 0)
            pltpu.sync_copy(src_v, out_hbm)

    return k


def _make_gather(n_table, d, n_rows, n_chunks):
    rpw = n_rows // _NW
    chunk = rpw // n_chunks
    assert rpw % n_chunks == 0 and chunk % 8 == 0
    nbuf = min(3, n_chunks)
    mesh = plsc.VectorSubcoreMesh(core_axis_name="c", subcore_axis_name="s")

    @functools.partial(
        pl.kernel,
        mesh=mesh,
        out_type=jax.ShapeDtypeStruct((n_rows, d), jnp.float32),
        scratch_types=[
            pltpu.VMEM((rpw,), jnp.int32),
            [pltpu.VMEM((chunk, d), jnp.float32) for _ in range(nbuf)],
            [pltpu.SemaphoreType.DMA for _ in range(nbuf)],
            [pltpu.SemaphoreType.DMA for _ in range(nbuf)],
        ],
        compiler_params=pltpu.CompilerParams(needs_layout_passes=False),
    )
    def k(table_hbm, idx_hbm, out_hbm, idx_v, bufs, gsems, wsems):
        wid = lax.axis_index("c") * _NS + lax.axis_index("s")
        base = wid * rpw
        pltpu.sync_copy(idx_hbm.at[pl.ds(base, rpw)], idx_v)

        def start_gather(c):
            return pltpu.async_copy(
                table_hbm.at[idx_v.at[pl.ds(c * chunk, chunk)]],
                bufs[c % nbuf], gsems[c % nbuf])

        ghandles = [None] * n_chunks
        whandles = [None] * n_chunks
        for c in range(min(nbuf - 1, n_chunks)):
            ghandles[c] = start_gather(c)
        for c in range(n_chunks):
            pre = c + nbuf - 1
            if pre < n_chunks:
                if pre - nbuf >= 0:
                    whandles[pre - nbuf].wait()
                ghandles[pre] = start_gather(pre)
            ghandles[c].wait()
            whandles[c] = pltpu.async_copy(
                bufs[c % nbuf], out_hbm.at[pl.ds(base + c * chunk, chunk)],
                wsems[c % nbuf])
        for c in range(max(0, n_chunks - nbuf), n_chunks):
            whandles[c].wait()

    return k


def _ffn_body(bf, x_ref, w1_ref, b1_ref, w2_ref, b2_ref, out_ref, xb_ref):
    j = pl.program_id(1)

    @pl.when(j == 0)
    def _cvt():
        xb_ref[...] = x_ref[...].astype(jnp.bfloat16)

    h = jnp.dot(xb_ref[...], w1_ref[0], preferred_element_type=jnp.float32)
    h = jax.nn.gelu(h + b1_ref[0])
    contrib = jnp.dot(h.astype(jnp.bfloat16), w2_ref[0],
                      preferred_element_type=jnp.float32)

    @pl.when(j == 0)
    def _init():
        out_ref[...] = contrib + b2_ref[0]

    @pl.when(j != 0)
    def _acc():
        out_ref[...] = out_ref[...] + contrib


def _ffn(xe, w1, b1, w2, b2, capacity):
    n_experts, d_model, d_ff = w1.shape
    bf = 512
    body = functools.partial(_ffn_body, bf)
    b1r = b1.reshape(n_experts, 1, d_ff)
    b2r = b2.reshape(n_experts, 1, d_model)
    return pl.pallas_call(
        body,
        grid=(n_experts, d_ff // bf),
        in_specs=[
            pl.BlockSpec((capacity, d_model), lambda e, j: (e, 0)),
            pl.BlockSpec((1, d_model, bf), lambda e, j: (e, 0, j)),
            pl.BlockSpec((1, 1, bf), lambda e, j: (e, 0, j)),
            pl.BlockSpec((1, bf, d_model), lambda e, j: (e, j, 0)),
            pl.BlockSpec((1, 1, d_model), lambda e, j: (e, 0, 0)),
        ],
        out_specs=pl.BlockSpec((capacity, d_model), lambda e, j: (e, 0)),
        out_shape=jax.ShapeDtypeStruct((n_experts * capacity, d_model),
                                       jnp.float32),
        scratch_shapes=[pltpu.VMEM((capacity, d_model), jnp.bfloat16)],
        compiler_params=pltpu.CompilerParams(
            dimension_semantics=("parallel", "arbitrary")),
    )(xe, w1.astype(jnp.bfloat16), b1r, w2.astype(jnp.bfloat16), b2r)


def _combine_body(g_ref, w_ref, out_ref):
    g = g_ref[...]
    w = w_ref[...]
    out_ref[...] = g[:, 0, :] * w[:, 0:1] + g[:, 1, :] * w[:, 1:2]


def _combine(gathered3, gates2):
    n_tokens, k_top, d_model = gathered3.shape
    blk = 256
    return pl.pallas_call(
        _combine_body,
        grid=(n_tokens // blk,),
        in_specs=[
            pl.BlockSpec((blk, k_top, d_model), lambda i: (i, 0, 0)),
            pl.BlockSpec((blk, k_top), lambda i: (i, 0)),
        ],
        out_specs=pl.BlockSpec((blk, d_model), lambda i: (i, 0)),
        out_shape=jax.ShapeDtypeStruct((n_tokens, d_model), jnp.float32),
    )(gathered3, gates2)


def kernel(x, Wr, w1, b1, w2, b2):
    B, T, d_model = x.shape
    n_experts = Wr.shape[1]
    k_top = 2
    n_tokens = B * T
    n_pairs = n_tokens * k_top
    capacity = math.ceil(1.25 * n_pairs / n_experts)
    n_slots = n_experts * capacity

    x_flat = x.reshape(n_tokens, d_model)
    slots2, gates2, valid2, aux = _router(x_flat, Wr, capacity, k_top)
    slots_flat = slots2.reshape(-1)
    valid_flat = valid2.reshape(-1)

    src_tok = _make_scatter_src(n_pairs, n_slots)(slots_flat, valid_flat)
    xe = _make_gather(n_tokens, d_model, n_slots, 4)(x_flat, src_tok)
    out_e = _ffn(xe, w1, b1, w2, b2, capacity)
    rows = _make_gather(n_slots, d_model, n_pairs, 4)(out_e, slots_flat)
    y_flat = _combine(rows.reshape(n_tokens, k_top, d_model), gates2)
    return y_flat.reshape(B, T, d_model), aux.reshape(())


# speedup vs baseline: 1.5036x; 1.5036x over previous
import functools
import math

import jax
import jax.numpy as jnp
from jax import lax
from jax.experimental import pallas as pl
from jax.experimental.pallas import tpu as pltpu
from jax.experimental.pallas import tpu_sc as plsc

_NC = 2
_NS = 16
_NW = _NC * _NS


def _router_body(n_tokens, n_experts, capacity, k_top,
                 x_ref, wr_ref, slots_ref, slots_cb_ref, gates_ref, aux_ref):
    f32 = jnp.float32
    xb = x_ref[...]
    wr = wr_ref[...]
    logits = jnp.dot(xb, wr, preferred_element_type=f32)
    m = jnp.max(logits, axis=1, keepdims=True)
    ex = jnp.exp(logits - m)
    probs = ex / jnp.sum(ex, axis=1, keepdims=True)

    eidx = lax.broadcasted_iota(jnp.int32, (n_tokens, n_experts), 1)
    m1 = jnp.max(probs, axis=1, keepdims=True)
    i1 = jnp.min(jnp.where(probs == m1, eidx, n_experts), axis=1,
                 keepdims=True)
    probs_m = jnp.where(eidx == i1, -jnp.inf, probs)
    m2 = jnp.max(probs_m, axis=1, keepdims=True)
    i2 = jnp.min(jnp.where(probs_m == m2, eidx, n_experts), axis=1,
                 keepdims=True)
    ssum = m1 + m2
    g1 = m1 / ssum
    g2 = m2 / ssum

    oh = (eidx == i1).astype(f32) + (eidx == i2).astype(f32)

    me = jnp.sum(probs, axis=0, keepdims=True) / n_tokens
    ce = jnp.sum(oh, axis=0, keepdims=True) / n_tokens
    aux_ref[...] = ((n_experts / k_top) * jnp.sum(me * ce)).reshape(1, 1)

    blk = 256
    cnt_rows = []
    for b in range(n_tokens // blk):
        r = lax.broadcasted_iota(jnp.int32, (blk, n_tokens), 0) + (b * blk)
        c = lax.broadcasted_iota(jnp.int32, (blk, n_tokens), 1)
        trib = (c < r).astype(f32)
        cnt_rows.append(jnp.dot(trib, oh, preferred_element_type=f32))
    cnt = jnp.concatenate(cnt_rows, axis=0)

    p1 = jnp.sum(jnp.where(eidx == i1, cnt, 0.0), axis=1, keepdims=True)
    p2 = jnp.sum(jnp.where(eidx == i2, cnt, 0.0), axis=1, keepdims=True)
    p1i = p1.astype(jnp.int32)
    p2i = p2.astype(jnp.int32)
    v1 = p1i < capacity
    v2 = p2i < capacity
    slot1 = i1 * capacity + p1i
    slot2 = i2 * capacity + p2i
    slots_ref[...] = jnp.concatenate(
        [jnp.where(v1, slot1, -1), jnp.where(v2, slot2, -1)], axis=1)
    slots_cb_ref[...] = jnp.concatenate(
        [jnp.where(v1, slot1, 0), jnp.where(v2, slot2, 0)], axis=1)
    gates_ref[...] = jnp.concatenate(
        [jnp.where(v1, g1, 0.0), jnp.where(v2, g2, 0.0)], axis=1)


def _router(x_flat, Wr, capacity, k_top):
    n_tokens, _ = x_flat.shape
    n_experts = Wr.shape[1]
    body = functools.partial(_router_body, n_tokens, n_experts, capacity,
                             k_top)
    return pl.pallas_call(
        body,
        out_shape=[
            jax.ShapeDtypeStruct((n_tokens, k_top), jnp.int32),
            jax.ShapeDtypeStruct((n_tokens, k_top), jnp.int32),
            jax.ShapeDtypeStruct((n_tokens, k_top), jnp.float32),
            jax.ShapeDtypeStruct((1, 1), jnp.float32),
        ],
    )(x_flat, Wr)


def _make_dispatch(n_tokens, d, n_pairs, n_slots, n_chunks):
    rpw = n_slots // _NW
    chunk = rpw // n_chunks
    assert rpw % n_chunks == 0 and chunk % 8 == 0 and rpw % 16 == 0
    nbuf = min(3, n_chunks)
    mesh = plsc.VectorSubcoreMesh(core_axis_name="c", subcore_axis_name="s")

    @functools.partial(
        pl.kernel,
        mesh=mesh,
        out_type=jax.ShapeDtypeStruct((n_slots, d), jnp.float32),
        scratch_types=[
            pltpu.VMEM((n_pairs,), jnp.int32),
            pltpu.VMEM((rpw,), jnp.int32),
            [pltpu.VMEM((chunk, d), jnp.float32) for _ in range(nbuf)],
            [pltpu.SemaphoreType.DMA for _ in range(nbuf)],
            [pltpu.SemaphoreType.DMA for _ in range(nbuf)],
        ],
        compiler_params=pltpu.CompilerParams(needs_layout_passes=False),
    )
    def k(x_hbm, slots_hbm, out_hbm, slots_v, src_v, bufs, gsems, wsems):
        wid = lax.axis_index("c") * _NS + lax.axis_index("s")
        base = wid * rpw
        pltpu.sync_copy(slots_hbm, slots_v)
        for i in range(rpw // 16):
            ids = base + i * 16 + lax.broadcasted_iota(jnp.int32, (16,), 0)
            src_v[pl.ds(i * 16, 16)] = lax.rem(ids, n_tokens)

        def sbody(j, carry):
            sl = slots_v[pl.ds(j * 16, 16)]
            vm = (sl >= base) & (sl < base + rpw)
            pair_ids = j * 16 + lax.broadcasted_iota(jnp.int32, (16,), 0)
            toks = lax.shift_right_logical(pair_ids, 1)
            plsc.store_scatter(src_v, [sl - base], toks, mask=vm)
            return carry

        lax.fori_loop(0, n_pairs // 16, sbody, 0)

        def start_gather(c):
            return pltpu.async_copy(
                x_hbm.at[src_v.at[pl.ds(c * chunk, chunk)]],
                bufs[c % nbuf], gsems[c % nbuf])

        ghandles = [None] * n_chunks
        whandles = [None] * n_chunks
        for c in range(min(nbuf - 1, n_chunks)):
            ghandles[c] = start_gather(c)
        for c in range(n_chunks):
            pre = c + nbuf - 1
            if pre < n_chunks:
                if pre - nbuf >= 0:
                    whandles[pre - nbuf].wait()
                ghandles[pre] = start_gather(pre)
            ghandles[c].wait()
            whandles[c] = pltpu.async_copy(
                bufs[c % nbuf], out_hbm.at[pl.ds(base + c * chunk, chunk)],
                wsems[c % nbuf])
        for c in range(max(0, n_chunks - nbuf), n_chunks):
            whandles[c].wait()

    return k


def _make_gather(n_table, d, n_rows, n_chunks):
    rpw = n_rows // _NW
    chunk = rpw // n_chunks
    assert rpw % n_chunks == 0 and chunk % 8 == 0
    nbuf = min(3, n_chunks)
    mesh = plsc.VectorSubcoreMesh(core_axis_name="c", subcore_axis_name="s")

    @functools.partial(
        pl.kernel,
        mesh=mesh,
        out_type=jax.ShapeDtypeStruct((n_rows, d), jnp.float32),
        scratch_types=[
            pltpu.VMEM((rpw,), jnp.int32),
            [pltpu.VMEM((chunk, d), jnp.float32) for _ in range(nbuf)],
            [pltpu.SemaphoreType.DMA for _ in range(nbuf)],
            [pltpu.SemaphoreType.DMA for _ in range(nbuf)],
        ],
        compiler_params=pltpu.CompilerParams(needs_layout_passes=False),
    )
    def k(table_hbm, idx_hbm, out_hbm, idx_v, bufs, gsems, wsems):
        wid = lax.axis_index("c") * _NS + lax.axis_index("s")
        base = wid * rpw
        pltpu.sync_copy(idx_hbm.at[pl.ds(base, rpw)], idx_v)

        def start_gather(c):
            return pltpu.async_copy(
                table_hbm.at[idx_v.at[pl.ds(c * chunk, chunk)]],
                bufs[c % nbuf], gsems[c % nbuf])

        ghandles = [None] * n_chunks
        whandles = [None] * n_chunks
        for c in range(min(nbuf - 1, n_chunks)):
            ghandles[c] = start_gather(c)
        for c in range(n_chunks):
            pre = c + nbuf - 1
            if pre < n_chunks:
                if pre - nbuf >= 0:
                    whandles[pre - nbuf].wait()
                ghandles[pre] = start_gather(pre)
            ghandles[c].wait()
            whandles[c] = pltpu.async_copy(
                bufs[c % nbuf], out_hbm.at[pl.ds(base + c * chunk, chunk)],
                wsems[c % nbuf])
        for c in range(max(0, n_chunks - nbuf), n_chunks):
            whandles[c].wait()

    return k


def _ffn_body(bf, x_ref, w1_ref, b1_ref, w2_ref, b2_ref, out_ref):
    j = pl.program_id(1)
    h = jnp.dot(x_ref[...], w1_ref[0], preferred_element_type=jnp.float32)
    h = jax.nn.gelu(h + b1_ref[0])
    contrib = jnp.dot(h, w2_ref[0], preferred_element_type=jnp.float32)

    @pl.when(j == 0)
    def _init():
        out_ref[...] = contrib + b2_ref[0]

    @pl.when(j != 0)
    def _acc():
        out_ref[...] = out_ref[...] + contrib


def _ffn(xe, w1, b1, w2, b2, capacity):
    n_experts, d_model, d_ff = w1.shape
    bf = 512
    body = functools.partial(_ffn_body, bf)
    b1r = b1.reshape(n_experts, 1, d_ff)
    b2r = b2.reshape(n_experts, 1, d_model)
    return pl.pallas_call(
        body,
        grid=(n_experts, d_ff // bf),
        in_specs=[
            pl.BlockSpec((capacity, d_model), lambda e, j: (e, 0)),
            pl.BlockSpec((1, d_model, bf), lambda e, j: (e, 0, j)),
            pl.BlockSpec((1, 1, bf), lambda e, j: (e, 0, j)),
            pl.BlockSpec((1, bf, d_model), lambda e, j: (e, j, 0)),
            pl.BlockSpec((1, 1, d_model), lambda e, j: (e, 0, 0)),
        ],
        out_specs=pl.BlockSpec((capacity, d_model), lambda e, j: (e, 0)),
        out_shape=jax.ShapeDtypeStruct((n_experts * capacity, d_model),
                                       jnp.float32),
        compiler_params=pltpu.CompilerParams(
            dimension_semantics=("parallel", "arbitrary")),
    )(xe, w1, b1r, w2, b2r)


def _combine_body(g_ref, w_ref, out_ref):
    g = g_ref[...]
    w = w_ref[...]
    out_ref[...] = g[:, 0, :] * w[:, 0:1] + g[:, 1, :] * w[:, 1:2]


def _combine(gathered3, gates2):
    n_tokens, k_top, d_model = gathered3.shape
    blk = 256
    return pl.pallas_call(
        _combine_body,
        grid=(n_tokens // blk,),
        in_specs=[
            pl.BlockSpec((blk, k_top, d_model), lambda i: (i, 0, 0)),
            pl.BlockSpec((blk, k_top), lambda i: (i, 0)),
        ],
        out_specs=pl.BlockSpec((blk, d_model), lambda i: (i, 0)),
        out_shape=jax.ShapeDtypeStruct((n_tokens, d_model), jnp.float32),
    )(gathered3, gates2)


def kernel(x, Wr, w1, b1, w2, b2):
    B, T, d_model = x.shape
    n_experts = Wr.shape[1]
    k_top = 2
    n_tokens = B * T
    n_pairs = n_tokens * k_top
    capacity = math.ceil(1.25 * n_pairs / n_experts)
    n_slots = n_experts * capacity

    x_flat = x.reshape(n_tokens, d_model)
    slots2, slots_cb2, gates2, aux = _router(x_flat, Wr, capacity, k_top)

    xe = _make_dispatch(n_tokens, d_model, n_pairs, n_slots, 4)(
        x_flat, slots2.reshape(-1))
    out_e = _ffn(xe, w1, b1, w2, b2, capacity)
    rows = _make_gather(n_slots, d_model, n_pairs, 4)(
        out_e, slots_cb2.reshape(-1))
    y_flat = _combine(rows.reshape(n_tokens, k_top, d_model), gates2)
    return y_flat.reshape(B, T, d_model), aux.reshape(())


# speedup vs baseline: 1.5540x; 1.0335x over previous
import functools
import math

import jax
import jax.numpy as jnp
from jax import lax
from jax.experimental import pallas as pl
from jax.experimental.pallas import tpu as pltpu
from jax.experimental.pallas import tpu_sc as plsc

_NC = 2
_NS = 16
_NW = _NC * _NS


def _router_body(n_tokens, n_experts, capacity, k_top,
                 x_ref, wr_ref, slots_ref, slots_cb_ref, gates_ref, aux_ref):
    f32 = jnp.float32
    xb = x_ref[...]
    wr = wr_ref[...]
    logits = jnp.dot(xb, wr, preferred_element_type=f32)
    m = jnp.max(logits, axis=1, keepdims=True)
    ex = jnp.exp(logits - m)
    probs = ex / jnp.sum(ex, axis=1, keepdims=True)

    eidx = lax.broadcasted_iota(jnp.int32, (n_tokens, n_experts), 1)
    m1 = jnp.max(probs, axis=1, keepdims=True)
    i1 = jnp.min(jnp.where(probs == m1, eidx, n_experts), axis=1,
                 keepdims=True)
    probs_m = jnp.where(eidx == i1, -jnp.inf, probs)
    m2 = jnp.max(probs_m, axis=1, keepdims=True)
    i2 = jnp.min(jnp.where(probs_m == m2, eidx, n_experts), axis=1,
                 keepdims=True)
    ssum = m1 + m2
    g1 = m1 / ssum
    g2 = m2 / ssum

    oh = (eidx == i1).astype(f32) + (eidx == i2).astype(f32)

    me = jnp.sum(probs, axis=0, keepdims=True) / n_tokens
    ce = jnp.sum(oh, axis=0, keepdims=True) / n_tokens
    aux_ref[...] = ((n_experts / k_top) * jnp.sum(me * ce)).reshape(1, 1)

    blk = 256
    cnt_rows = []
    for b in range(n_tokens // blk):
        r = lax.broadcasted_iota(jnp.int32, (blk, n_tokens), 0) + (b * blk)
        c = lax.broadcasted_iota(jnp.int32, (blk, n_tokens), 1)
        trib = (c < r).astype(f32)
        cnt_rows.append(jnp.dot(trib, oh, preferred_element_type=f32))
    cnt = jnp.concatenate(cnt_rows, axis=0)

    p1 = jnp.sum(jnp.where(eidx == i1, cnt, 0.0), axis=1, keepdims=True)
    p2 = jnp.sum(jnp.where(eidx == i2, cnt, 0.0), axis=1, keepdims=True)
    p1i = p1.astype(jnp.int32)
    p2i = p2.astype(jnp.int32)
    v1 = p1i < capacity
    v2 = p2i < capacity
    slot1 = i1 * capacity + p1i
    slot2 = i2 * capacity + p2i
    slots_ref[...] = jnp.concatenate(
        [jnp.where(v1, slot1, -1), jnp.where(v2, slot2, -1)], axis=1)
    counts = jnp.sum(oh, axis=0, keepdims=True)
    fill = jnp.minimum(counts, float(capacity))
    eidx_row = lax.broadcasted_iota(jnp.int32, (1, n_experts), 1)
    e_dump = jnp.min(jnp.where(fill < capacity, eidx_row, n_experts))
    fill_dump = jnp.sum(jnp.where(eidx_row == e_dump, fill, 0.0))
    dump_slot = e_dump * capacity + fill_dump.astype(jnp.int32)
    slots_cb_ref[...] = jnp.concatenate(
        [jnp.where(v1, slot1, dump_slot), jnp.where(v2, slot2, dump_slot)],
        axis=1)
    gates_ref[...] = jnp.concatenate(
        [jnp.where(v1, g1, 0.0), jnp.where(v2, g2, 0.0)], axis=1)


def _router(x_flat, Wr, capacity, k_top):
    n_tokens, _ = x_flat.shape
    n_experts = Wr.shape[1]
    body = functools.partial(_router_body, n_tokens, n_experts, capacity,
                             k_top)
    return pl.pallas_call(
        body,
        out_shape=[
            jax.ShapeDtypeStruct((n_tokens, k_top), jnp.int32),
            jax.ShapeDtypeStruct((n_tokens, k_top), jnp.int32),
            jax.ShapeDtypeStruct((n_tokens, k_top), jnp.float32),
            jax.ShapeDtypeStruct((1, 1), jnp.float32),
        ],
    )(x_flat, Wr)


def _make_dispatch(n_tokens, d, n_pairs, n_slots, n_chunks):
    rpw = n_slots // _NW
    chunk = rpw // n_chunks
    assert rpw % n_chunks == 0 and chunk % 8 == 0 and rpw % 16 == 0
    nbuf = min(3, n_chunks)
    mesh = plsc.VectorSubcoreMesh(core_axis_name="c", subcore_axis_name="s")

    @functools.partial(
        pl.kernel,
        mesh=mesh,
        out_type=[
            jax.ShapeDtypeStruct((n_slots, d), jnp.float32),
            jax.ShapeDtypeStruct((n_slots,), jnp.float32),
        ],
        scratch_types=[
            pltpu.VMEM((n_pairs,), jnp.int32),
            pltpu.VMEM((n_pairs,), jnp.float32),
            pltpu.VMEM((rpw,), jnp.int32),
            pltpu.VMEM((rpw,), jnp.float32),
            [pltpu.VMEM((chunk, d), jnp.float32) for _ in range(nbuf)],
            [pltpu.SemaphoreType.DMA for _ in range(nbuf)],
            [pltpu.SemaphoreType.DMA for _ in range(nbuf)],
        ],
        compiler_params=pltpu.CompilerParams(needs_layout_passes=False),
    )
    def k(x_hbm, slots_hbm, gates_hbm, out_hbm, gate_out_hbm,
          slots_v, gates_v, src_v, gate_v, bufs, gsems, wsems):
        wid = lax.axis_index("c") * _NS + lax.axis_index("s")
        base = wid * rpw
        pltpu.sync_copy(slots_hbm, slots_v)
        pltpu.sync_copy(gates_hbm, gates_v)
        for i in range(rpw // 16):
            ids = base + i * 16 + lax.broadcasted_iota(jnp.int32, (16,), 0)
            src_v[pl.ds(i * 16, 16)] = lax.rem(ids, n_tokens)
            gate_v[pl.ds(i * 16, 16)] = jnp.zeros((16,), jnp.float32)

        def sbody(j, carry):
            sl = slots_v[pl.ds(j * 16, 16)]
            vm = (sl >= base) & (sl < base + rpw)
            pair_ids = j * 16 + lax.broadcasted_iota(jnp.int32, (16,), 0)
            toks = lax.shift_right_logical(pair_ids, 1)
            plsc.store_scatter(src_v, [sl - base], toks, mask=vm)
            plsc.store_scatter(gate_v, [sl - base],
                               gates_v[pl.ds(j * 16, 16)], mask=vm)
            return carry

        lax.fori_loop(0, n_pairs // 16, sbody, 0)
        pltpu.sync_copy(gate_v, gate_out_hbm.at[pl.ds(base, rpw)])

        def start_gather(c):
            return pltpu.async_copy(
                x_hbm.at[src_v.at[pl.ds(c * chunk, chunk)]],
                bufs[c % nbuf], gsems[c % nbuf])

        ghandles = [None] * n_chunks
        whandles = [None] * n_chunks
        for c in range(min(nbuf - 1, n_chunks)):
            ghandles[c] = start_gather(c)
        for c in range(n_chunks):
            pre = c + nbuf - 1
            if pre < n_chunks:
                if pre - nbuf >= 0:
                    whandles[pre - nbuf].wait()
                ghandles[pre] = start_gather(pre)
            ghandles[c].wait()
            whandles[c] = pltpu.async_copy(
                bufs[c % nbuf], out_hbm.at[pl.ds(base + c * chunk, chunk)],
                wsems[c % nbuf])
        for c in range(max(0, n_chunks - nbuf), n_chunks):
            whandles[c].wait()

    return k


def _ffn_body(n_j, x_ref, g_ref, w1_ref, b1_ref, w2_ref, b2_ref, out_ref):
    j = pl.program_id(1)
    h = jnp.dot(x_ref[...], w1_ref[0], preferred_element_type=jnp.float32)
    h = jax.nn.gelu(h + b1_ref[0])
    contrib = jnp.dot(h, w2_ref[0], preferred_element_type=jnp.float32)

    @pl.when(j == 0)
    def _init():
        out_ref[...] = contrib

    @pl.when(jnp.logical_and(j > 0, j < n_j - 1))
    def _acc():
        out_ref[...] = out_ref[...] + contrib

    @pl.when(j == n_j - 1)
    def _fin():
        out_ref[...] = (out_ref[...] + contrib + b2_ref[0]) * g_ref[0]


def _ffn(xe, gate_slot, w1, b1, w2, b2, capacity):
    n_experts, d_model, d_ff = w1.shape
    bf = 512
    body = functools.partial(_ffn_body, d_ff // bf)
    b1r = b1.reshape(n_experts, 1, d_ff)
    b2r = b2.reshape(n_experts, 1, d_model)
    g3 = gate_slot.reshape(n_experts, capacity, 1)
    return pl.pallas_call(
        body,
        grid=(n_experts, d_ff // bf),
        in_specs=[
            pl.BlockSpec((capacity, d_model), lambda e, j: (e, 0)),
            pl.BlockSpec((1, capacity, 1), lambda e, j: (e, 0, 0)),
            pl.BlockSpec((1, d_model, bf), lambda e, j: (e, 0, j)),
            pl.BlockSpec((1, 1, bf), lambda e, j: (e, 0, j)),
            pl.BlockSpec((1, bf, d_model), lambda e, j: (e, j, 0)),
            pl.BlockSpec((1, 1, d_model), lambda e, j: (e, 0, 0)),
        ],
        out_specs=pl.BlockSpec((capacity, d_model), lambda e, j: (e, 0)),
        out_shape=jax.ShapeDtypeStruct((n_experts * capacity, d_model),
                                       jnp.float32),
        compiler_params=pltpu.CompilerParams(
            dimension_semantics=("parallel", "arbitrary")),
    )(xe, g3, w1, b1r, w2, b2r)


def _make_combine(n_slots, d, n_tokens, n_chunks):
    tpw = n_tokens // _NW
    ch_tok = tpw // n_chunks
    ch_pairs = 2 * ch_tok
    assert tpw % n_chunks == 0 and ch_pairs % 8 == 0
    mesh = plsc.VectorSubcoreMesh(core_axis_name="c", subcore_axis_name="s")

    @functools.partial(
        pl.kernel,
        mesh=mesh,
        out_type=jax.ShapeDtypeStruct((n_tokens, d), jnp.float32),
        scratch_types=[
            pltpu.VMEM((2 * tpw,), jnp.int32),
            [pltpu.VMEM((ch_pairs, d), jnp.float32) for _ in range(2)],
            pltpu.VMEM((ch_tok, d), jnp.float32),
            [pltpu.SemaphoreType.DMA for _ in range(2)],
        ],
        compiler_params=pltpu.CompilerParams(needs_layout_passes=False),
    )
    def k(table_hbm, idx_hbm, y_hbm, idx_v, bufs, ybuf, gsems):
        wid = lax.axis_index("c") * _NS + lax.axis_index("s")
        tok0 = wid * tpw
        pltpu.sync_copy(idx_hbm.at[pl.ds(2 * tok0, 2 * tpw)], idx_v)

        def start_gather(c):
            return pltpu.async_copy(
                table_hbm.at[idx_v.at[pl.ds(c * ch_pairs, ch_pairs)]],
                bufs[c % 2], gsems[c % 2])

        handles = [None] * n_chunks
        handles[0] = start_gather(0)
        for c in range(n_chunks):
            if c + 1 < n_chunks:
                handles[c + 1] = start_gather(c + 1)
            handles[c].wait()
            buf = bufs[c % 2]

            def tbody(t, carry):
                for v in range(d // 16):
                    r0 = buf[2 * t, pl.ds(v * 16, 16)]
                    r1 = buf[2 * t + 1, pl.ds(v * 16, 16)]
                    ybuf[t, pl.ds(v * 16, 16)] = r0 + r1
                return carry

            lax.fori_loop(0, ch_tok, tbody, 0)
            pltpu.sync_copy(ybuf, y_hbm.at[pl.ds(tok0 + c * ch_tok, ch_tok)])

    return k


def kernel(x, Wr, w1, b1, w2, b2):
    B, T, d_model = x.shape
    n_experts = Wr.shape[1]
    k_top = 2
    n_tokens = B * T
    n_pairs = n_tokens * k_top
    capacity = math.ceil(1.25 * n_pairs / n_experts)
    n_slots = n_experts * capacity

    x_flat = x.reshape(n_tokens, d_model)
    slots2, slots_cb2, gates2, aux = _router(x_flat, Wr, capacity, k_top)

    xe, gate_slot = _make_dispatch(n_tokens, d_model, n_pairs, n_slots, 5)(
        x_flat, slots2.reshape(-1), gates2.reshape(-1))
    out_e = _ffn(xe, gate_slot, w1, b1, w2, b2, capacity)
    y_flat = _make_combine(n_slots, d_model, n_tokens, 4)(
        out_e, slots_cb2.reshape(-1))
    return y_flat.reshape(B, T, d_model), aux.reshape(())


# speedup vs baseline: 1.7479x; 1.1248x over previous
import functools
import math

import jax
import jax.numpy as jnp
from jax import lax
from jax.experimental import pallas as pl
from jax.experimental.pallas import tpu as pltpu
from jax.experimental.pallas import tpu_sc as plsc

_NC = 2
_NS = 16
_NW = _NC * _NS


def _router_body(n_tokens, n_experts, capacity, k_top,
                 x_ref, wr_ref, slots_ref, slots_cb_ref, gates_ref, aux_ref):
    f32 = jnp.float32
    xb = x_ref[...]
    wr = wr_ref[...]
    logits = jnp.dot(xb, wr, preferred_element_type=f32)
    m = jnp.max(logits, axis=1, keepdims=True)
    ex = jnp.exp(logits - m)
    probs = ex / jnp.sum(ex, axis=1, keepdims=True)

    eidx = lax.broadcasted_iota(jnp.int32, (n_tokens, n_experts), 1)
    m1 = jnp.max(probs, axis=1, keepdims=True)
    i1 = jnp.min(jnp.where(probs == m1, eidx, n_experts), axis=1,
                 keepdims=True)
    probs_m = jnp.where(eidx == i1, -jnp.inf, probs)
    m2 = jnp.max(probs_m, axis=1, keepdims=True)
    i2 = jnp.min(jnp.where(probs_m == m2, eidx, n_experts), axis=1,
                 keepdims=True)
    ssum = m1 + m2
    g1 = m1 / ssum
    g2 = m2 / ssum

    oh = (eidx == i1).astype(f32) + (eidx == i2).astype(f32)

    me = jnp.sum(probs, axis=0, keepdims=True) / n_tokens
    ce = jnp.sum(oh, axis=0, keepdims=True) / n_tokens
    aux_ref[...] = ((n_experts / k_top) * jnp.sum(me * ce)).reshape(1, 1)

    blk = 256
    cnt_rows = []
    for b in range(n_tokens // blk):
        r = lax.broadcasted_iota(jnp.int32, (blk, n_tokens), 0) + (b * blk)
        c = lax.broadcasted_iota(jnp.int32, (blk, n_tokens), 1)
        trib = (c < r).astype(f32)
        cnt_rows.append(jnp.dot(trib, oh, preferred_element_type=f32))
    cnt = jnp.concatenate(cnt_rows, axis=0)

    p1 = jnp.sum(jnp.where(eidx == i1, cnt, 0.0), axis=1, keepdims=True)
    p2 = jnp.sum(jnp.where(eidx == i2, cnt, 0.0), axis=1, keepdims=True)
    p1i = p1.astype(jnp.int32)
    p2i = p2.astype(jnp.int32)
    v1 = p1i < capacity
    v2 = p2i < capacity
    slot1 = i1 * capacity + p1i
    slot2 = i2 * capacity + p2i
    slots_ref[...] = jnp.concatenate(
        [jnp.where(v1, slot1, -1), jnp.where(v2, slot2, -1)], axis=1)
    counts = jnp.sum(oh, axis=0, keepdims=True)
    fill = jnp.minimum(counts, float(capacity))
    eidx_row = lax.broadcasted_iota(jnp.int32, (1, n_experts), 1)
    e_dump = jnp.min(jnp.where(fill < capacity, eidx_row, n_experts))
    fill_dump = jnp.sum(jnp.where(eidx_row == e_dump, fill, 0.0))
    dump_slot = e_dump * capacity + fill_dump.astype(jnp.int32)
    slots_cb_ref[...] = jnp.concatenate(
        [jnp.where(v1, slot1, dump_slot), jnp.where(v2, slot2, dump_slot)],
        axis=1)
    gates_ref[...] = jnp.concatenate(
        [jnp.where(v1, g1, 0.0), jnp.where(v2, g2, 0.0)], axis=1)


def _router(x_flat, Wr, capacity, k_top):
    n_tokens, _ = x_flat.shape
    n_experts = Wr.shape[1]
    body = functools.partial(_router_body, n_tokens, n_experts, capacity,
                             k_top)
    return pl.pallas_call(
        body,
        out_shape=[
            jax.ShapeDtypeStruct((n_tokens, k_top), jnp.int32),
            jax.ShapeDtypeStruct((n_tokens, k_top), jnp.int32),
            jax.ShapeDtypeStruct((n_tokens, k_top), jnp.float32),
            jax.ShapeDtypeStruct((1, 1), jnp.float32),
        ],
    )(x_flat, Wr)


def _make_dispatch(n_tokens, d, n_pairs, n_slots, n_chunks):
    rpw = n_slots // _NW
    chunk = rpw // n_chunks
    assert rpw % n_chunks == 0 and chunk % 8 == 0 and rpw % 16 == 0
    nbuf = min(3, n_chunks)
    mesh = plsc.VectorSubcoreMesh(core_axis_name="c", subcore_axis_name="s")

    @functools.partial(
        pl.kernel,
        mesh=mesh,
        out_type=[
            jax.ShapeDtypeStruct((n_slots, d), jnp.float32),
            jax.ShapeDtypeStruct((n_slots,), jnp.float32),
        ],
        scratch_types=[
            pltpu.VMEM((n_pairs,), jnp.int32),
            pltpu.VMEM((n_pairs,), jnp.float32),
            pltpu.VMEM((rpw,), jnp.int32),
            pltpu.VMEM((rpw,), jnp.float32),
            [pltpu.VMEM((chunk, d), jnp.float32) for _ in range(nbuf)],
            [pltpu.SemaphoreType.DMA for _ in range(nbuf)],
            [pltpu.SemaphoreType.DMA for _ in range(nbuf)],
        ],
        compiler_params=pltpu.CompilerParams(needs_layout_passes=False),
    )
    def k(x_hbm, slots_hbm, gates_hbm, out_hbm, gate_out_hbm,
          slots_v, gates_v, src_v, gate_v, bufs, gsems, wsems):
        wid = lax.axis_index("c") * _NS + lax.axis_index("s")
        base = wid * rpw
        pltpu.sync_copy(slots_hbm, slots_v)
        pltpu.sync_copy(gates_hbm, gates_v)
        for i in range(rpw // 16):
            ids = base + i * 16 + lax.broadcasted_iota(jnp.int32, (16,), 0)
            src_v[pl.ds(i * 16, 16)] = lax.rem(ids, n_tokens)
            gate_v[pl.ds(i * 16, 16)] = jnp.zeros((16,), jnp.float32)

        def sbody(j, carry):
            sl = slots_v[pl.ds(j * 16, 16)]
            vm = (sl >= base) & (sl < base + rpw)
            pair_ids = j * 16 + lax.broadcasted_iota(jnp.int32, (16,), 0)
            toks = lax.shift_right_logical(pair_ids, 1)
            plsc.store_scatter(src_v, [sl - base], toks, mask=vm)
            plsc.store_scatter(gate_v, [sl - base],
                               gates_v[pl.ds(j * 16, 16)], mask=vm)
            return carry

        lax.fori_loop(0, n_pairs // 16, sbody, 0)
        pltpu.sync_copy(gate_v, gate_out_hbm.at[pl.ds(base, rpw)])

        def start_gather(c):
            return pltpu.async_copy(
                x_hbm.at[src_v.at[pl.ds(c * chunk, chunk)]],
                bufs[c % nbuf], gsems[c % nbuf])

        ghandles = [None] * n_chunks
        whandles = [None] * n_chunks
        for c in range(min(nbuf - 1, n_chunks)):
            ghandles[c] = start_gather(c)
        for c in range(n_chunks):
            pre = c + nbuf - 1
            if pre < n_chunks:
                if pre - nbuf >= 0:
                    whandles[pre - nbuf].wait()
                ghandles[pre] = start_gather(pre)
            ghandles[c].wait()
            whandles[c] = pltpu.async_copy(
                bufs[c % nbuf], out_hbm.at[pl.ds(base + c * chunk, chunk)],
                wsems[c % nbuf])
        for c in range(max(0, n_chunks - nbuf), n_chunks):
            whandles[c].wait()

    return k


def _ffn_body(n_j, x_ref, g_ref, w1_ref, b1_ref, w2_ref, b2_ref, out_ref):
    j = pl.program_id(1)
    h = jnp.dot(x_ref[...], w1_ref[0], preferred_element_type=jnp.float32)
    h = jax.nn.gelu(h + b1_ref[0])
    contrib = jnp.dot(h, w2_ref[0], preferred_element_type=jnp.float32)

    @pl.when(j == 0)
    def _init():
        out_ref[...] = contrib

    @pl.when(jnp.logical_and(j > 0, j < n_j - 1))
    def _acc():
        out_ref[...] = out_ref[...] + contrib

    @pl.when(j == n_j - 1)
    def _fin():
        out_ref[...] = (out_ref[...] + contrib + b2_ref[0]) * g_ref[0]


def _ffn(xe, gate_slot, w1, b1, w2, b2, capacity):
    n_experts, d_model, d_ff = w1.shape
    bf = 1024
    body = functools.partial(_ffn_body, d_ff // bf)
    b1r = b1.reshape(n_experts, 1, d_ff)
    b2r = b2.reshape(n_experts, 1, d_model)
    g3 = gate_slot.reshape(n_experts, capacity, 1)
    return pl.pallas_call(
        body,
        grid=(n_experts, d_ff // bf),
        in_specs=[
            pl.BlockSpec((capacity, d_model), lambda e, j: (e, 0)),
            pl.BlockSpec((1, capacity, 1), lambda e, j: (e, 0, 0)),
            pl.BlockSpec((1, d_model, bf), lambda e, j: (e, 0, j)),
            pl.BlockSpec((1, 1, bf), lambda e, j: (e, 0, j)),
            pl.BlockSpec((1, bf, d_model), lambda e, j: (e, j, 0)),
            pl.BlockSpec((1, 1, d_model), lambda e, j: (e, 0, 0)),
        ],
        out_specs=pl.BlockSpec((capacity, d_model), lambda e, j: (e, 0)),
        out_shape=jax.ShapeDtypeStruct((n_experts * capacity, d_model),
                                       jnp.float32),
        compiler_params=pltpu.CompilerParams(
            dimension_semantics=("parallel", "arbitrary")),
    )(xe, g3, w1, b1r, w2, b2r)


def _make_combine(n_slots, d, n_tokens, n_chunks):
    tpw = n_tokens // _NW
    ch_tok = tpw // n_chunks
    ch_pairs = 2 * ch_tok
    assert tpw % n_chunks == 0 and ch_pairs % 8 == 0
    mesh = plsc.VectorSubcoreMesh(core_axis_name="c", subcore_axis_name="s")

    @functools.partial(
        pl.kernel,
        mesh=mesh,
        out_type=jax.ShapeDtypeStruct((n_tokens, d), jnp.float32),
        scratch_types=[
            pltpu.VMEM((2 * tpw,), jnp.int32),
            [pltpu.VMEM((ch_pairs, d), jnp.float32) for _ in range(2)],
            pltpu.VMEM((ch_tok, d), jnp.float32),
            [pltpu.SemaphoreType.DMA for _ in range(2)],
        ],
        compiler_params=pltpu.CompilerParams(needs_layout_passes=False),
    )
    def k(table_hbm, idx_hbm, y_hbm, idx_v, bufs, ybuf, gsems):
        wid = lax.axis_index("c") * _NS + lax.axis_index("s")
        tok0 = wid * tpw
        pltpu.sync_copy(idx_hbm.at[pl.ds(2 * tok0, 2 * tpw)], idx_v)

        def start_gather(c):
            return pltpu.async_copy(
                table_hbm.at[idx_v.at[pl.ds(c * ch_pairs, ch_pairs)]],
                bufs[c % 2], gsems[c % 2])

        handles = [None] * n_chunks
        handles[0] = start_gather(0)
        for c in range(n_chunks):
            if c + 1 < n_chunks:
                handles[c + 1] = start_gather(c + 1)
            handles[c].wait()
            buf = bufs[c % 2]

            def tbody(t, carry):
                for v in range(d // 16):
                    r0 = buf[2 * t, pl.ds(v * 16, 16)]
                    r1 = buf[2 * t + 1, pl.ds(v * 16, 16)]
                    ybuf[t, pl.ds(v * 16, 16)] = r0 + r1
                return carry

            lax.fori_loop(0, ch_tok, tbody, 0)
            pltpu.sync_copy(ybuf, y_hbm.at[pl.ds(tok0 + c * ch_tok, ch_tok)])

    return k


def kernel(x, Wr, w1, b1, w2, b2):
    B, T, d_model = x.shape
    n_experts = Wr.shape[1]
    k_top = 2
    n_tokens = B * T
    n_pairs = n_tokens * k_top
    capacity = math.ceil(1.25 * n_pairs / n_experts)
    n_slots = n_experts * capacity

    x_flat = x.reshape(n_tokens, d_model)
    slots2, slots_cb2, gates2, aux = _router(x_flat, Wr, capacity, k_top)

    xe, gate_slot = _make_dispatch(n_tokens, d_model, n_pairs, n_slots, 5)(
        x_flat, slots2.reshape(-1), gates2.reshape(-1))
    out_e = _ffn(xe, gate_slot, w1, b1, w2, b2, capacity)
    y_flat = _make_combine(n_slots, d_model, n_tokens, 4)(
        out_e, slots_cb2.reshape(-1))
    return y_flat.reshape(B, T, d_model), aux.reshape(())


# speedup vs baseline: 1.8322x; 1.0483x over previous
import functools
import math

import jax
import jax.numpy as jnp
from jax import lax
from jax.experimental import pallas as pl
from jax.experimental.pallas import tpu as pltpu
from jax.experimental.pallas import tpu_sc as plsc

_NC = 2
_NS = 16
_NW = _NC * _NS


def _router_body(n_tokens, n_experts, capacity, k_top,
                 x_ref, wr_ref, slots_ref, slots_cb_ref, gates_ref, aux_ref):
    f32 = jnp.float32
    xb = x_ref[...]
    wr = wr_ref[...]
    logits = jnp.dot(xb, wr, preferred_element_type=f32)
    m = jnp.max(logits, axis=1, keepdims=True)
    ex = jnp.exp(logits - m)
    probs = ex / jnp.sum(ex, axis=1, keepdims=True)

    eidx = lax.broadcasted_iota(jnp.int32, (n_tokens, n_experts), 1)
    m1 = jnp.max(probs, axis=1, keepdims=True)
    i1 = jnp.min(jnp.where(probs == m1, eidx, n_experts), axis=1,
                 keepdims=True)
    probs_m = jnp.where(eidx == i1, -jnp.inf, probs)
    m2 = jnp.max(probs_m, axis=1, keepdims=True)
    i2 = jnp.min(jnp.where(probs_m == m2, eidx, n_experts), axis=1,
                 keepdims=True)
    ssum = m1 + m2
    g1 = m1 / ssum
    g2 = m2 / ssum

    oh = (eidx == i1).astype(f32) + (eidx == i2).astype(f32)

    me = jnp.sum(probs, axis=0, keepdims=True) / n_tokens
    ce = jnp.sum(oh, axis=0, keepdims=True) / n_tokens
    aux_ref[...] = ((n_experts / k_top) * jnp.sum(me * ce)).reshape(1, 1)

    blk = 256
    cnt_rows = []
    for b in range(n_tokens // blk):
        r = lax.broadcasted_iota(jnp.int32, (blk, n_tokens), 0) + (b * blk)
        c = lax.broadcasted_iota(jnp.int32, (blk, n_tokens), 1)
        trib = (c < r).astype(f32)
        cnt_rows.append(jnp.dot(trib, oh, preferred_element_type=f32))
    cnt = jnp.concatenate(cnt_rows, axis=0)

    p1 = jnp.sum(jnp.where(eidx == i1, cnt, 0.0), axis=1, keepdims=True)
    p2 = jnp.sum(jnp.where(eidx == i2, cnt, 0.0), axis=1, keepdims=True)
    p1i = p1.astype(jnp.int32)
    p2i = p2.astype(jnp.int32)
    v1 = p1i < capacity
    v2 = p2i < capacity
    slot1 = i1 * capacity + p1i
    slot2 = i2 * capacity + p2i
    slots_ref[...] = jnp.concatenate(
        [jnp.where(v1, slot1, -1), jnp.where(v2, slot2, -1)], axis=1)
    counts = jnp.sum(oh, axis=0, keepdims=True)
    fill = jnp.minimum(counts, float(capacity))
    eidx_row = lax.broadcasted_iota(jnp.int32, (1, n_experts), 1)
    e_dump = jnp.min(jnp.where(fill < capacity, eidx_row, n_experts))
    fill_dump = jnp.sum(jnp.where(eidx_row == e_dump, fill, 0.0))
    dump_slot = e_dump * capacity + fill_dump.astype(jnp.int32)
    slots_cb_ref[...] = jnp.concatenate(
        [jnp.where(v1, slot1, dump_slot), jnp.where(v2, slot2, dump_slot)],
        axis=1)
    gates_ref[...] = jnp.concatenate(
        [jnp.where(v1, g1, 0.0), jnp.where(v2, g2, 0.0)], axis=1)


def _router(x_flat, Wr, capacity, k_top):
    n_tokens, _ = x_flat.shape
    n_experts = Wr.shape[1]
    body = functools.partial(_router_body, n_tokens, n_experts, capacity,
                             k_top)
    return pl.pallas_call(
        body,
        out_shape=[
            jax.ShapeDtypeStruct((n_tokens, k_top), jnp.int32),
            jax.ShapeDtypeStruct((n_tokens, k_top), jnp.int32),
            jax.ShapeDtypeStruct((n_tokens, k_top), jnp.float32),
            jax.ShapeDtypeStruct((1, 1), jnp.float32),
        ],
    )(x_flat, Wr)


def _make_dispatch(n_tokens, d, n_pairs, n_slots, n_chunks):
    rpw = n_slots // _NW
    chunk = rpw // n_chunks
    assert rpw % n_chunks == 0 and chunk % 8 == 0 and rpw % 16 == 0
    nbuf = min(3, n_chunks)
    mesh = plsc.VectorSubcoreMesh(core_axis_name="c", subcore_axis_name="s")

    @functools.partial(
        pl.kernel,
        mesh=mesh,
        out_type=[
            jax.ShapeDtypeStruct((n_slots, d), jnp.float32),
            jax.ShapeDtypeStruct((n_slots,), jnp.float32),
        ],
        scratch_types=[
            pltpu.VMEM((n_pairs,), jnp.int32),
            pltpu.VMEM((n_pairs,), jnp.float32),
            pltpu.VMEM((rpw,), jnp.int32),
            pltpu.VMEM((rpw,), jnp.float32),
            [pltpu.VMEM((chunk, d), jnp.float32) for _ in range(nbuf)],
            [pltpu.SemaphoreType.DMA for _ in range(nbuf)],
            [pltpu.SemaphoreType.DMA for _ in range(nbuf)],
        ],
        compiler_params=pltpu.CompilerParams(needs_layout_passes=False),
    )
    def k(x_hbm, slots_hbm, gates_hbm, out_hbm, gate_out_hbm,
          slots_v, gates_v, src_v, gate_v, bufs, gsems, wsems):
        wid = lax.axis_index("c") * _NS + lax.axis_index("s")
        base = wid * rpw
        pltpu.sync_copy(slots_hbm, slots_v)
        pltpu.sync_copy(gates_hbm, gates_v)
        for i in range(rpw // 16):
            ids = base + i * 16 + lax.broadcasted_iota(jnp.int32, (16,), 0)
            src_v[pl.ds(i * 16, 16)] = lax.rem(ids, n_tokens)
            gate_v[pl.ds(i * 16, 16)] = jnp.zeros((16,), jnp.float32)

        def sbody(j, carry):
            sl = slots_v[pl.ds(j * 16, 16)]
            vm = (sl >= base) & (sl < base + rpw)
            pair_ids = j * 16 + lax.broadcasted_iota(jnp.int32, (16,), 0)
            toks = lax.shift_right_logical(pair_ids, 1)
            plsc.store_scatter(src_v, [sl - base], toks, mask=vm)
            plsc.store_scatter(gate_v, [sl - base],
                               gates_v[pl.ds(j * 16, 16)], mask=vm)
            return carry

        lax.fori_loop(0, n_pairs // 16, sbody, 0)
        pltpu.sync_copy(gate_v, gate_out_hbm.at[pl.ds(base, rpw)])

        def start_gather(c):
            return pltpu.async_copy(
                x_hbm.at[src_v.at[pl.ds(c * chunk, chunk)]],
                bufs[c % nbuf], gsems[c % nbuf])

        ghandles = [None] * n_chunks
        whandles = [None] * n_chunks
        for c in range(min(nbuf - 1, n_chunks)):
            ghandles[c] = start_gather(c)
        for c in range(n_chunks):
            pre = c + nbuf - 1
            if pre < n_chunks:
                if pre - nbuf >= 0:
                    whandles[pre - nbuf].wait()
                ghandles[pre] = start_gather(pre)
            ghandles[c].wait()
            whandles[c] = pltpu.async_copy(
                bufs[c % nbuf], out_hbm.at[pl.ds(base + c * chunk, chunk)],
                wsems[c % nbuf])
        for c in range(max(0, n_chunks - nbuf), n_chunks):
            whandles[c].wait()

    return k


def _ffn_body(n_j, x_ref, g_ref, w1_ref, b1_ref, w2_ref, b2_ref, out_ref):
    j = pl.program_id(1)
    h = jnp.dot(x_ref[...], w1_ref[0], preferred_element_type=jnp.float32)
    h = jax.nn.gelu(h + b1_ref[0])
    contrib = jnp.dot(h, w2_ref[0], preferred_element_type=jnp.float32)

    @pl.when(j == 0)
    def _init():
        out_ref[...] = contrib

    @pl.when(jnp.logical_and(j > 0, j < n_j - 1))
    def _acc():
        out_ref[...] = out_ref[...] + contrib

    @pl.when(j == n_j - 1)
    def _fin():
        out_ref[...] = (out_ref[...] + contrib + b2_ref[0]) * g_ref[0]


def _ffn(xe, gate_slot, w1, b1, w2, b2, capacity):
    n_experts, d_model, d_ff = w1.shape
    bf = 2048
    body = functools.partial(_ffn_body, d_ff // bf)
    b1r = b1.reshape(n_experts, 1, d_ff)
    b2r = b2.reshape(n_experts, 1, d_model)
    g3 = gate_slot.reshape(n_experts, capacity, 1)
    return pl.pallas_call(
        body,
        grid=(n_experts, d_ff // bf),
        in_specs=[
            pl.BlockSpec((capacity, d_model), lambda e, j: (e, 0)),
            pl.BlockSpec((1, capacity, 1), lambda e, j: (e, 0, 0)),
            pl.BlockSpec((1, d_model, bf), lambda e, j: (e, 0, j)),
            pl.BlockSpec((1, 1, bf), lambda e, j: (e, 0, j)),
            pl.BlockSpec((1, bf, d_model), lambda e, j: (e, j, 0)),
            pl.BlockSpec((1, 1, d_model), lambda e, j: (e, 0, 0)),
        ],
        out_specs=pl.BlockSpec((capacity, d_model), lambda e, j: (e, 0)),
        out_shape=jax.ShapeDtypeStruct((n_experts * capacity, d_model),
                                       jnp.float32),
        compiler_params=pltpu.CompilerParams(
            dimension_semantics=("parallel", "arbitrary")),
    )(xe, g3, w1, b1r, w2, b2r)


def _make_combine(n_slots, d, n_tokens, n_chunks):
    tpw = n_tokens // _NW
    ch_tok = tpw // n_chunks
    ch_pairs = 2 * ch_tok
    assert tpw % n_chunks == 0 and ch_pairs % 8 == 0
    mesh = plsc.VectorSubcoreMesh(core_axis_name="c", subcore_axis_name="s")

    @functools.partial(
        pl.kernel,
        mesh=mesh,
        out_type=jax.ShapeDtypeStruct((n_tokens, d), jnp.float32),
        scratch_types=[
            pltpu.VMEM((2 * tpw,), jnp.int32),
            [pltpu.VMEM((ch_pairs, d), jnp.float32) for _ in range(2)],
            pltpu.VMEM((ch_tok, d), jnp.float32),
            [pltpu.SemaphoreType.DMA for _ in range(2)],
        ],
        compiler_params=pltpu.CompilerParams(needs_layout_passes=False),
    )
    def k(table_hbm, idx_hbm, y_hbm, idx_v, bufs, ybuf, gsems):
        wid = lax.axis_index("c") * _NS + lax.axis_index("s")
        tok0 = wid * tpw
        pltpu.sync_copy(idx_hbm.at[pl.ds(2 * tok0, 2 * tpw)], idx_v)

        def start_gather(c):
            return pltpu.async_copy(
                table_hbm.at[idx_v.at[pl.ds(c * ch_pairs, ch_pairs)]],
                bufs[c % 2], gsems[c % 2])

        handles = [None] * n_chunks
        handles[0] = start_gather(0)
        for c in range(n_chunks):
            if c + 1 < n_chunks:
                handles[c + 1] = start_gather(c + 1)
            handles[c].wait()
            buf = bufs[c % 2]

            def tbody(t, carry):
                for v in range(d // 16):
                    r0 = buf[2 * t, pl.ds(v * 16, 16)]
                    r1 = buf[2 * t + 1, pl.ds(v * 16, 16)]
                    ybuf[t, pl.ds(v * 16, 16)] = r0 + r1
                return carry

            lax.fori_loop(0, ch_tok, tbody, 0)
            pltpu.sync_copy(ybuf, y_hbm.at[pl.ds(tok0 + c * ch_tok, ch_tok)])

    return k


def kernel(x, Wr, w1, b1, w2, b2):
    B, T, d_model = x.shape
    n_experts = Wr.shape[1]
    k_top = 2
    n_tokens = B * T
    n_pairs = n_tokens * k_top
    capacity = math.ceil(1.25 * n_pairs / n_experts)
    n_slots = n_experts * capacity

    x_flat = x.reshape(n_tokens, d_model)
    slots2, slots_cb2, gates2, aux = _router(x_flat, Wr, capacity, k_top)

    xe, gate_slot = _make_dispatch(n_tokens, d_model, n_pairs, n_slots, 5)(
        x_flat, slots2.reshape(-1), gates2.reshape(-1))
    out_e = _ffn(xe, gate_slot, w1, b1, w2, b2, capacity)
    y_flat = _make_combine(n_slots, d_model, n_tokens, 4)(
        out_e, slots_cb2.reshape(-1))
    return y_flat.reshape(B, T, d_model), aux.reshape(())
